# baseline (speedup 1.0000x reference)
import jax
import jax.numpy as jnp
from jax import lax
from jax.experimental import pallas as pl
from jax.experimental.pallas import tpu as pltpu
from jax.experimental.pallas import tpu_sc as plsc

_B = 16384
_L = 16
_NC = 2
_NS = 16
_NW = _NC * _NS
_BPW = _B // _NW
_C = 64
_NCH = _BPW // _C
_G = _C // _L
_MARGIN = 1.0
_UNROLL = 4
_D = 64
_W = 128
_ER = 4 * _C


def _transe_body(xT, ent, rel2, out, idxb, gidxb, colb, e0, e1, r0, r1,
                 obuf, sem_x, sem0, sem1):
    wid = lax.axis_index("s") * _NC + lax.axis_index("c")
    base = wid * _BPW

    cps = [pltpu.async_copy(xT.at[j, pl.ds(base, _BPW)], idxb.at[j], sem_x)
           for j in range(6)]
    for cp in cps:
        cp.wait()

    iota = lax.iota(jnp.int32, _L)

    def tbody(g, _):
        sl = pl.ds(g * _L, _L)
        for j, src in ((0, 2), (1, 5)):
            v = idxb[src, sl]
            gidxb[j, sl] = lax.shift_right_logical(v, 1)
            colb[j, sl] = lax.shift_left(v & 1, 6)
        return 0

    lax.fori_loop(0, _BPW // _L, tbody, 0)

    ebufs = (e0, e1)
    rbufs = (r0, r1)
    sems = (sem0, sem1)

    def fire(k, s):
        sl = pl.ds(k * _C, _C)
        eb, rb, sm = ebufs[s], rbufs[s], sems[s]
        rel_cps = [
            pltpu.async_copy(rel2.at[gidxb.at[0, sl]], rb.at[0], sm),
            pltpu.async_copy(rel2.at[gidxb.at[1, sl]], rb.at[1], sm),
        ]

        def issue(g, _, eb=eb, sm=sm, k=k):
            for j, role in enumerate((0, 1, 3, 4)):
                v = idxb[role, pl.ds(k * _C + g * _L, _L)]
                rowbase = j * _C + g * _L
                for lane in range(_L):
                    e = v[lane]
                    pltpu.async_copy(ent.at[pl.ds(e, 1)],
                                     eb.at[pl.ds(rowbase + lane, 1)], sm)
            return 0

        lax.fori_loop(0, _G, issue, 0)
        return rel_cps

    def drain(s):
        pltpu.make_async_copy(ent.at[pl.ds(0, _ER)], ebufs[s], sems[s]).wait()

    acc = jnp.zeros((_L,), jnp.float32)
    cps = fire(0, 0)
    for k in range(_NCH):
        nxt = fire(k + 1, (k + 1) % 2) if k + 1 < _NCH else None
        for cp in cps:
            cp.wait()
        drain(k % 2)
        eb, rb = ebufs[k % 2], rbufs[k % 2]

        def gbody(g, acc, eb=eb, rb=rb, k=k):
            rows = g * _L + iota
            gsl = pl.ds(k * _C + g * _L, _L)
            cb = [colb[j, gsl] for j in range(2)]

            def cbody(ci, rowsum, eb=eb, rb=rb, cb=cb, rows=rows):
                for u in range(_UNROLL):
                    c = ci * _UNROLL + u
                    col = jnp.full((_L,), c, jnp.int32)
                    a = plsc.load_gather(eb, [rows, col])
                    t = plsc.load_gather(eb, [_C + rows, col])
                    d = plsc.load_gather(eb, [2 * _C + rows, col])
                    e = plsc.load_gather(eb, [3 * _C + rows, col])
                    b = plsc.load_gather(rb.at[0], [rows, cb[0] + c])
                    f = plsc.load_gather(rb.at[1], [rows, cb[1] + c])
                    rowsum = rowsum + (jnp.abs(a + b - t) - jnp.abs(d + f - e))
                return rowsum

            rowsum = lax.fori_loop(0, _D // _UNROLL, cbody,
                                   jnp.zeros((_L,), jnp.float32))
            return acc + jnp.maximum(rowsum + _MARGIN, 0.0)

        acc = lax.fori_loop(0, _G, gbody, acc)
        cps = nxt

    obuf[...] = acc
    pltpu.sync_copy(obuf, out.at[wid])


def _transe_partials(xT, ent_emb, rel2):
    f32 = jnp.float32
    run = pl.kernel(
        _transe_body,
        mesh=plsc.VectorSubcoreMesh(core_axis_name="c", subcore_axis_name="s"),
        compiler_params=pltpu.CompilerParams(
            needs_layout_passes=False, use_tc_tiling_on_sc=True),
        out_type=jax.ShapeDtypeStruct((_NW, _L), f32),
        scratch_types=[
            pltpu.VMEM((6, _BPW), jnp.int32),
            pltpu.VMEM((2, _BPW), jnp.int32),
            pltpu.VMEM((2, _BPW), jnp.int32),
            pltpu.VMEM((_ER, _D), f32),
            pltpu.VMEM((_ER, _D), f32),
            pltpu.VMEM((2, _C, _W), f32),
            pltpu.VMEM((2, _C, _W), f32),
            pltpu.VMEM((_L,), f32),
            pltpu.SemaphoreType.DMA,
            pltpu.SemaphoreType.DMA,
            pltpu.SemaphoreType.DMA,
        ],
    )
    return run(xT, ent_emb, rel2)


def kernel(x, ent_emb, rel_emb):
    xT = x.T
    rel2 = rel_emb.reshape(500, _W)
    partials = _transe_partials(xT, ent_emb, rel2)
    probe = jnp.take(ent_emb, x[:8, 0], axis=0)
    return jnp.sum(partials) + 0.0 * jnp.sum(probe)


# speedup vs baseline: 1.3246x; 1.3246x over previous
import jax
import jax.numpy as jnp
from jax import lax
from jax.experimental import pallas as pl
from jax.experimental.pallas import tpu as pltpu
from jax.experimental.pallas import tpu_sc as plsc

_B = 16384
_L = 16
_NC = 2
_NS = 16
_NW = _NC * _NS
_BPW = _B // _NW
_C = 64
_NCH = _BPW // _C
_G = _C // _L
_MARGIN = 1.0
_UNROLL = 4
_D = 64
_W = 128
_ER = 4 * _C


def _transe_body(xT, ent, rel2, out, idxb, gidxb, colb, e0, e1, r0, r1,
                 obuf, sem_x, sem0, sem1):
    wid = lax.axis_index("s") * _NC + lax.axis_index("c")
    base = wid * _BPW

    cps = [pltpu.async_copy(xT.at[j, pl.ds(base, _BPW)], idxb.at[j], sem_x)
           for j in range(6)]
    for cp in cps:
        cp.wait()

    iota = lax.iota(jnp.int32, _L)

    def tbody(g, _):
        sl = pl.ds(g * _L, _L)
        for j, src in ((0, 2), (1, 5)):
            v = idxb[src, sl]
            gidxb[j, sl] = lax.shift_right_logical(v, 1)
            colb[j, sl] = lax.shift_left(v & 1, 6)
        return 0

    lax.fori_loop(0, _BPW // _L, tbody, 0)

    ebufs = (e0, e1)
    rbufs = (r0, r1)
    sems = (sem0, sem1)

    def fire(k, s):
        sl = pl.ds(k * _C, _C)
        eb, rb, sm = ebufs[s], rbufs[s], sems[s]
        rel_cps = [
            pltpu.async_copy(rel2.at[gidxb.at[0, sl]], rb.at[0], sm),
            pltpu.async_copy(rel2.at[gidxb.at[1, sl]], rb.at[1], sm),
        ]

        def issue(g, _, eb=eb, sm=sm, k=k):
            for j, role in enumerate((0, 1, 3, 4)):
                v = idxb[role, pl.ds(k * _C + g * _L, _L)]
                rowbase = j * _C + g * _L
                for lane in range(_L):
                    e = v[lane]
                    q = jnp.where(e < 500000, 0, 1).astype(jnp.int32)
                    r = e - q * 500000
                    pltpu.async_copy(ent.at[q, pl.ds(r, 1)],
                                     eb.at[pl.ds(rowbase + lane, 1)], sm)
            return 0

        lax.fori_loop(0, _G, issue, 0)
        return rel_cps

    def drain(s):
        pltpu.make_async_copy(ent.at[0, pl.ds(0, _ER)], ebufs[s],
                              sems[s]).wait()

    acc = jnp.zeros((_L,), jnp.float32)
    cps = fire(0, 0)
    for k in range(_NCH):
        nxt = fire(k + 1, (k + 1) % 2) if k + 1 < _NCH else None
        for cp in cps:
            cp.wait()
        drain(k % 2)
        eb, rb = ebufs[k % 2], rbufs[k % 2]

        def gbody(g, acc, eb=eb, rb=rb, k=k):
            rows = g * _L + iota
            gsl = pl.ds(k * _C + g * _L, _L)
            cb = [colb[j, gsl] for j in range(2)]

            def cbody(ci, rowsum, eb=eb, rb=rb, cb=cb, rows=rows):
                for u in range(_UNROLL):
                    c = ci * _UNROLL + u
                    col = jnp.full((_L,), c, jnp.int32)
                    a = plsc.load_gather(eb, [rows, col])
                    t = plsc.load_gather(eb, [_C + rows, col])
                    d = plsc.load_gather(eb, [2 * _C + rows, col])
                    e = plsc.load_gather(eb, [3 * _C + rows, col])
                    b = plsc.load_gather(rb.at[0], [rows, cb[0] + c])
                    f = plsc.load_gather(rb.at[1], [rows, cb[1] + c])
                    rowsum = rowsum + (jnp.abs(a + b - t) - jnp.abs(d + f - e))
                return rowsum

            rowsum = lax.fori_loop(0, _D // _UNROLL, cbody,
                                   jnp.zeros((_L,), jnp.float32))
            return acc + jnp.maximum(rowsum + _MARGIN, 0.0)

        acc = lax.fori_loop(0, _G, gbody, acc)
        cps = nxt

    obuf[...] = acc
    pltpu.sync_copy(obuf, out.at[wid])


def _transe_partials(xT, ent_emb, rel2):
    f32 = jnp.float32
    run = pl.kernel(
        _transe_body,
        mesh=plsc.VectorSubcoreMesh(core_axis_name="c", subcore_axis_name="s"),
        compiler_params=pltpu.CompilerParams(
            needs_layout_passes=False, use_tc_tiling_on_sc=True),
        out_type=jax.ShapeDtypeStruct((_NW, _L), f32),
        scratch_types=[
            pltpu.VMEM((6, _BPW), jnp.int32),
            pltpu.VMEM((2, _BPW), jnp.int32),
            pltpu.VMEM((2, _BPW), jnp.int32),
            pltpu.VMEM((_ER, _D), f32),
            pltpu.VMEM((_ER, _D), f32),
            pltpu.VMEM((2, _C, _W), f32),
            pltpu.VMEM((2, _C, _W), f32),
            pltpu.VMEM((_L,), f32),
            pltpu.SemaphoreType.DMA,
            pltpu.SemaphoreType.DMA,
            pltpu.SemaphoreType.DMA,
        ],
    )
    return run(xT, ent_emb, rel2)


def kernel(x, ent_emb, rel_emb):
    xT = x.T
    ent3 = ent_emb.reshape(2, 500000, _D)
    rel2 = rel_emb.reshape(500, _W)
    partials = _transe_partials(xT, ent3, rel2)
    return jnp.sum(partials)


# speedup vs baseline: 1.7388x; 1.3127x over previous
import jax
import jax.numpy as jnp
from jax import lax
from jax.experimental import pallas as pl
from jax.experimental.pallas import tpu as pltpu
from jax.experimental.pallas import tpu_sc as plsc

_B = 16384
_L = 16
_NC = 2
_NS = 16
_NW = _NC * _NS
_BPW = _B // _NW
_C = 64
_NCH = _BPW // _C
_G = _C // _L
_MARGIN = 1.0
_D = 64
_SR = 6 * _C
_HALF = 500000


def _transe_body(xT, ent, rel, out, idxb, s0, s1, obuf, sem_x, sem0, sem1):
    wid = lax.axis_index("s") * _NC + lax.axis_index("c")
    base = wid * _BPW

    cps = [pltpu.async_copy(xT.at[j, pl.ds(base, _BPW)], idxb.at[j], sem_x)
           for j in range(6)]
    for cp in cps:
        cp.wait()

    iota = lax.iota(jnp.int32, _L)
    sbufs = (s0, s1)
    sems = (sem0, sem1)

    def fire(k, s):
        sb, sm = sbufs[s], sems[s]

        def issue(g, _, sb=sb, sm=sm, k=k):
            sl = pl.ds(k * _C + g * _L, _L)
            for j, role in enumerate((0, 1, 3, 4)):
                v = idxb[role, sl]
                rowbase = j * _C + g * _L
                for lane in range(_L):
                    e = v[lane]
                    q = jnp.where(e < _HALF, 0, 1).astype(jnp.int32)
                    r = e - q * _HALF
                    pltpu.async_copy(ent.at[q, pl.ds(r, 1)],
                                     sb.at[pl.ds(rowbase + lane, 1)], sm)
            for j, role in enumerate((2, 5)):
                v = idxb[role, sl]
                rowbase = (4 + j) * _C + g * _L
                for lane in range(_L):
                    pltpu.async_copy(rel.at[pl.ds(v[lane], 1)],
                                     sb.at[pl.ds(rowbase + lane, 1)], sm)
            return 0

        lax.fori_loop(0, _G, issue, 0)

    def drain(s):
        pltpu.make_async_copy(ent.at[0, pl.ds(0, _SR)], sbufs[s],
                              sems[s]).wait()

    acc = jnp.float32(0.0)
    fire(0, 0)
    for k in range(_NCH):
        if k + 1 < _NCH:
            fire(k + 1, (k + 1) % 2)
        drain(k % 2)
        sb = sbufs[k % 2]

        def tbody(i, acc, sb=sb):
            diff = jnp.zeros((_L,), jnp.float32)
            for u in range(_D // _L):
                sl = pl.ds(u * _L, _L)
                ph = sb[i, sl]
                pt = sb[_C + i, sl]
                nh = sb[2 * _C + i, sl]
                nt = sb[3 * _C + i, sl]
                pr = sb[4 * _C + i, sl]
                nr = sb[5 * _C + i, sl]
                diff = diff + (jnp.abs(ph + pr - pt) - jnp.abs(nh + nr - nt))
            return acc + jnp.maximum(jnp.sum(diff) + _MARGIN, 0.0)

        acc = lax.fori_loop(0, _C, tbody, acc)

    obuf[...] = jnp.where(iota == 0, acc, 0.0)
    pltpu.sync_copy(obuf, out.at[wid])


def _transe_partials(xT, ent3, rel):
    f32 = jnp.float32
    run = pl.kernel(
        _transe_body,
        mesh=plsc.VectorSubcoreMesh(core_axis_name="c", subcore_axis_name="s"),
        compiler_params=pltpu.CompilerParams(
            needs_layout_passes=False, use_tc_tiling_on_sc=True),
        out_type=jax.ShapeDtypeStruct((_NW, _L), f32),
        scratch_types=[
            pltpu.VMEM((6, _BPW), jnp.int32),
            pltpu.VMEM((_SR, _D), f32),
            pltpu.VMEM((_SR, _D), f32),
            pltpu.VMEM((_L,), f32),
            pltpu.SemaphoreType.DMA,
            pltpu.SemaphoreType.DMA,
            pltpu.SemaphoreType.DMA,
        ],
    )
    return run(xT, ent3, rel)


def kernel(x, ent_emb, rel_emb):
    xT = x.T
    ent3 = ent_emb.reshape(2, _HALF, _D)
    partials = _transe_partials(xT, ent3, rel_emb)
    return jnp.sum(partials)
